# baseline (speedup 1.0000x reference)
import jax
import jax.numpy as jnp
from jax.experimental import pallas as pl
from jax.experimental.pallas import tpu as pltpu

_ROWS = 128
_COLS = 100000
_SUP_STRIDE = 200
_SUP_LIMIT = 6400
_TAIL_CHUNKS = 8
_CHUNK_ROWS = _ROWS // _TAIL_CHUNKS


def _body(x_hbm, o_hbm, head_vmem, sem_in, sem_out, sem_tail):
    tail_cps = []
    for k in range(_TAIL_CHUNKS):
        r0 = k * _CHUNK_ROWS
        cp = pltpu.make_async_copy(
            x_hbm.at[pl.ds(r0, _CHUNK_ROWS), pl.ds(_SUP_LIMIT, _COLS - _SUP_LIMIT)],
            o_hbm.at[pl.ds(r0, _CHUNK_ROWS), pl.ds(_SUP_LIMIT, _COLS - _SUP_LIMIT)],
            sem_tail.at[k],
        )
        cp.start()
        tail_cps.append(cp)
    head_in = pltpu.make_async_copy(
        x_hbm.at[:, pl.ds(0, _SUP_LIMIT)], head_vmem, sem_in
    )
    head_in.start()
    head_in.wait()
    neg = jnp.full((_ROWS, 1), -jnp.inf, jnp.float32)
    for c in range(0, _SUP_LIMIT, _SUP_STRIDE):
        head_vmem[:, c : c + 1] = neg
    head_out = pltpu.make_async_copy(
        head_vmem, o_hbm.at[:, pl.ds(0, _SUP_LIMIT)], sem_out
    )
    head_out.start()
    head_out.wait()
    for cp in tail_cps:
        cp.wait()


def kernel(scores):
    return pl.pallas_call(
        _body,
        in_specs=[pl.BlockSpec(memory_space=pl.MemorySpace.ANY)],
        out_specs=pl.BlockSpec(memory_space=pl.MemorySpace.ANY),
        out_shape=jax.ShapeDtypeStruct((_ROWS, _COLS), scores.dtype),
        scratch_shapes=[
            pltpu.MemorySpace.VMEM(((_ROWS, _SUP_LIMIT)), jnp.float32),
            pltpu.SemaphoreType.DMA,
            pltpu.SemaphoreType.DMA,
            pltpu.SemaphoreType.DMA((_TAIL_CHUNKS,)),
        ],
    )(scores)
